# baseline (speedup 1.0000x reference)
import functools

import jax
import jax.numpy as jnp
from jax import lax
from jax.experimental import pallas as pl
from jax.experimental.pallas import tpu as pltpu
from jax.experimental.pallas import tpu_sc as plsc

_B, _C1, _C2, _H, _W = 4, 384, 384, 56, 56
_E, _K = 8, 2
_HW = _H * _W

_NW = 32
_WPB = _NW // _B
_RPW = _C2 // _WPB
_NCH = _RPW // 16


def _route_kernel(x_ref, Wr_ref, idx_ref, wts_ref):
    for b in range(_B):
        xb = x_ref[b]
        gap = jnp.mean(xb, axis=1, keepdims=True)
        logits = jnp.sum(gap * Wr_ref[...], axis=0, keepdims=True)
        iota = lax.broadcasted_iota(jnp.int32, (1, _E), 1)
        m1 = jnp.max(logits)
        i1 = jnp.min(jnp.where(logits == m1, iota, _E))
        masked = jnp.where(iota == i1, -jnp.inf, logits)
        m2 = jnp.max(masked)
        i2 = jnp.min(jnp.where(masked == m2, iota, _E))
        e = jnp.exp(m2 - m1)
        w0 = 1.0 / (1.0 + e)
        w1 = e / (1.0 + e)
        idx_ref[2 * b : 2 * b + 1, :] = jnp.full((1, 128), i1, jnp.int32)
        idx_ref[2 * b + 1 : 2 * b + 2, :] = jnp.full((1, 128), i2, jnp.int32)
        wts_ref[2 * b : 2 * b + 1, :] = jnp.full((1, 128), w0, jnp.float32)
        wts_ref[2 * b + 1 : 2 * b + 2, :] = jnp.full((1, 128), w1, jnp.float32)


def _route(x, Wr):
    return pl.pallas_call(
        _route_kernel,
        grid=(1,),
        in_specs=[
            pl.BlockSpec((_B, _C1, _HW), lambda i: (0, 0, 0)),
            pl.BlockSpec((_C1, _E), lambda i: (0, 0)),
        ],
        out_specs=[
            pl.BlockSpec((2 * _B, 128), lambda i: (0, 0)),
            pl.BlockSpec((2 * _B, 128), lambda i: (0, 0)),
        ],
        out_shape=[
            jax.ShapeDtypeStruct((2 * _B, 128), jnp.int32),
            jax.ShapeDtypeStruct((2 * _B, 128), jnp.float32),
        ],
        compiler_params=pltpu.CompilerParams(
            dimension_semantics=("arbitrary",),
            vmem_limit_bytes=100 * 1024 * 1024,
        ),
    )(x, Wr)


def _make_combine():
    mesh = plsc.VectorSubcoreMesh(core_axis_name="c", subcore_axis_name="s")

    @functools.partial(
        pl.kernel,
        mesh=mesh,
        out_type=jax.ShapeDtypeStruct((_B * _C2, _C1), jnp.float32),
        scratch_types=[
            pltpu.VMEM((2 * 16,), jnp.int32),
            pltpu.VMEM((2 * 16,), jnp.float32),
            pltpu.VMEM((16,), jnp.int32),
            pltpu.VMEM((16,), jnp.int32),
            pltpu.VMEM((16, _C1), jnp.float32),
            pltpu.VMEM((16, _C1), jnp.float32),
            pltpu.VMEM((16, _C1), jnp.float32),
            pltpu.VMEM((16, _C1), jnp.float32),
            pltpu.SemaphoreType.DMA,
            pltpu.SemaphoreType.DMA,
        ],
    )
    def combine(we_hbm, ws_hbm, idx_hbm, wts_hbm, wc_hbm,
                idx_v, wts_v, rid0_v, rid1_v, r0_v, r1_v, ws_v, acc_v,
                sem0, sem1):
        wid = lax.axis_index("s") * 2 + lax.axis_index("c")
        b = wid // _WPB
        chunk0 = (wid % _WPB) * _NCH
        pltpu.sync_copy(idx_hbm.at[pl.ds(2 * b * 128, 16)], idx_v.at[pl.ds(0, 16)])
        pltpu.sync_copy(
            idx_hbm.at[pl.ds((2 * b + 1) * 128, 16)], idx_v.at[pl.ds(16, 16)]
        )
        pltpu.sync_copy(wts_hbm.at[pl.ds(2 * b * 128, 16)], wts_v.at[pl.ds(0, 16)])
        pltpu.sync_copy(
            wts_hbm.at[pl.ds((2 * b + 1) * 128, 16)], wts_v.at[pl.ds(16, 16)]
        )
        lane = lax.iota(jnp.int32, 16)
        i0 = idx_v[pl.ds(0, 16)]
        i1 = idx_v[pl.ds(16, 16)]
        w0 = wts_v[pl.ds(0, 16)]
        w1 = wts_v[pl.ds(16, 16)]
        for c in range(_NCH):
            row0 = (chunk0 + c) * 16
            rows = row0 + lane
            rid0_v[...] = i0 * _C2 + rows
            rid1_v[...] = i1 * _C2 + rows
            cp0 = pltpu.async_copy(we_hbm.at[rid0_v], r0_v, sem0)
            cp1 = pltpu.async_copy(we_hbm.at[rid1_v], r1_v, sem1)
            pltpu.sync_copy(ws_hbm.at[pl.ds(row0, 16)], ws_v)
            cp0.wait()
            cp1.wait()

            def body(r, _):
                for cc in range(_C1 // 16):
                    sl = pl.ds(cc * 16, 16)
                    acc_v[r, sl] = (
                        w0 * r0_v[r, sl] + w1 * r1_v[r, sl] + ws_v[r, sl]
                    )
                return 0

            lax.fori_loop(0, 16, body, 0)
            pltpu.sync_copy(acc_v, wc_hbm.at[pl.ds(b * _C2 + row0, 16)])

    return combine


_combine = _make_combine()


def _mm_kernel(x_ref, wc_ref, out_ref):
    for s in range(2):
        xb = x_ref[s]
        out_ref[s] = jnp.dot(wc_ref[s], xb, preferred_element_type=jnp.float32) + xb


def _matmul(x, Wc):
    return pl.pallas_call(
        _mm_kernel,
        grid=(_B // 2,),
        in_specs=[
            pl.BlockSpec((2, _C1, _HW), lambda b: (b, 0, 0)),
            pl.BlockSpec((2, _C2, _C1), lambda b: (b, 0, 0)),
        ],
        out_specs=pl.BlockSpec((2, _C2, _HW), lambda b: (b, 0, 0)),
        out_shape=jax.ShapeDtypeStruct((_B, _C2, _HW), jnp.float32),
        compiler_params=pltpu.CompilerParams(
            dimension_semantics=("arbitrary",),
            vmem_limit_bytes=100 * 1024 * 1024,
        ),
    )(x, Wc)


def kernel(x, Wr, We, Ws):
    xr = x.reshape(_B, _C1, _HW)
    idx, wts = _route(xr, Wr)
    wc_flat = _combine(
        We.reshape(_E * _C2, _C1), Ws, idx.reshape(-1), wts.reshape(-1)
    )
    out = _matmul(xr, wc_flat.reshape(_B, _C2, _C1))
    return out.reshape(_B, _C2, _H, _W)


# speedup vs baseline: 1.6034x; 1.6034x over previous
import jax
import jax.numpy as jnp
from jax.experimental import pallas as pl
from jax.experimental.pallas import tpu as pltpu

_B, _C1, _C2, _H, _W = 4, 384, 384, 56, 56
_E, _K = 8, 2
_HW = _H * _W
_BS = 2


def _moe_kernel(x_ref, Wr_ref, We_ref, Ws_ref, out_ref):
    for s in range(_BS):
        xb = x_ref[s]
        gap = jnp.mean(xb, axis=1, keepdims=True)
        logits = jnp.sum(gap * Wr_ref[...], axis=0, keepdims=True)
        iota = jax.lax.broadcasted_iota(jnp.int32, (1, _E), 1)
        m1 = jnp.max(logits)
        i1 = jnp.min(jnp.where(logits == m1, iota, _E))
        masked = jnp.where(iota == i1, -jnp.inf, logits)
        m2 = jnp.max(masked)
        i2 = jnp.min(jnp.where(masked == m2, iota, _E))
        e = jnp.exp(m2 - m1)
        w0 = 1.0 / (1.0 + e)
        w1 = e / (1.0 + e)
        Wc = w0 * We_ref[i1] + w1 * We_ref[i2] + Ws_ref[...]
        out_ref[s] = jnp.dot(Wc, xb, preferred_element_type=jnp.float32) + xb


def kernel(x, Wr, We, Ws):
    xr = x.reshape(_B, _C1, _HW)
    out = pl.pallas_call(
        _moe_kernel,
        grid=(_B // _BS,),
        in_specs=[
            pl.BlockSpec((_BS, _C1, _HW), lambda b: (b, 0, 0)),
            pl.BlockSpec((_C1, _E), lambda b: (0, 0)),
            pl.BlockSpec((_E, _C2, _C1), lambda b: (0, 0, 0)),
            pl.BlockSpec((_C2, _C1), lambda b: (0, 0)),
        ],
        out_specs=pl.BlockSpec((_BS, _C2, _HW), lambda b: (b, 0, 0)),
        out_shape=jax.ShapeDtypeStruct((_B, _C2, _HW), jnp.float32),
        compiler_params=pltpu.CompilerParams(
            dimension_semantics=("arbitrary",),
            vmem_limit_bytes=100 * 1024 * 1024,
        ),
    )(xr, Wr, We, Ws)
    return out.reshape(_B, _C2, _H, _W)
